# baseline (speedup 1.0000x reference)
import jax
import jax.numpy as jnp
from jax.experimental import pallas as pl
from jax.experimental.pallas import tpu as pltpu

_ROWS = 320000
_GROUP = 48000
_CHUNK = 8000


def _make_read_group_body(base):
    def _read_group_body(x_ref, o_ref, sems):
        rows = o_ref.shape[0]
        nchunks = rows // _CHUNK

        def cp(i):
            return pltpu.make_async_copy(
                x_ref.at[pl.ds(base + i * _CHUNK, _CHUNK), :],
                o_ref.at[pl.ds(i * _CHUNK, _CHUNK), :], sems.at[i])

        for i in range(nchunks):
            cp(i).start()
        for i in range(nchunks):
            cp(i).wait()
    return _read_group_body


def _make_write_group_body(base):
    def _write_group_body(v_ref, prev_ref, o_ref, sems):
        del prev_ref
        rows = v_ref.shape[0]
        nchunks = rows // _CHUNK

        def cp(i):
            return pltpu.make_async_copy(
                v_ref.at[pl.ds(i * _CHUNK, _CHUNK), :],
                o_ref.at[pl.ds(base + i * _CHUNK, _CHUNK), :], sems.at[i])

        for i in range(nchunks):
            cp(i).start()
        for i in range(nchunks):
            cp(i).wait()
    return _write_group_body


def _read_group(x, base, rows):
    return pl.pallas_call(
        _make_read_group_body(base),
        in_specs=[pl.BlockSpec(memory_space=pltpu.MemorySpace.HBM)],
        out_specs=pl.BlockSpec(memory_space=pltpu.MemorySpace.VMEM),
        out_shape=jax.ShapeDtypeStruct((rows, x.shape[1]), x.dtype),
        scratch_shapes=[pltpu.SemaphoreType.DMA((rows // _CHUNK,))],
    )(x)


def _write_group(vbuf, prev, base):
    return pl.pallas_call(
        _make_write_group_body(base),
        in_specs=[pl.BlockSpec(memory_space=pltpu.MemorySpace.VMEM),
                  pl.BlockSpec(memory_space=pltpu.MemorySpace.HBM)],
        out_specs=pl.BlockSpec(memory_space=pltpu.MemorySpace.HBM),
        out_shape=jax.ShapeDtypeStruct(prev.shape, prev.dtype),
        input_output_aliases={1: 0},
        scratch_shapes=[pltpu.SemaphoreType.DMA((vbuf.shape[0] // _CHUNK,))],
    )(vbuf, prev)


def _small_read_body(*refs):
    n = len(refs) // 2
    ins, outs, sems = refs[:n], refs[n:2 * n], refs[2 * n]
    for i in range(n):
        pltpu.make_async_copy(ins[i], outs[i], sems.at[i]).start()
    for i in range(n):
        pltpu.make_async_copy(ins[i], outs[i], sems.at[i]).wait()


def kernel(node_features, edge_features, edges, node_hidden, edge_hidden,
           batch_indices, W1, W2, W3, U1, U2):
    hbm = pltpu.MemorySpace.HBM
    vmem = pltpu.MemorySpace.VMEM

    eh = None
    for base in range(0, _ROWS, _GROUP):
        rows = min(_GROUP, _ROWS - base)
        vbuf = _read_group(edge_hidden, base, rows)
        if eh is None:
            eh = pl.pallas_call(
                _make_write_group_body(0),
                in_specs=[pl.BlockSpec(memory_space=vmem),
                          pl.BlockSpec(memory_space=hbm)],
                out_specs=pl.BlockSpec(memory_space=hbm),
                out_shape=jax.ShapeDtypeStruct(edge_hidden.shape,
                                               edge_hidden.dtype),
                scratch_shapes=[pltpu.SemaphoreType.DMA((rows // _CHUNK,))],
            )(vbuf, edge_hidden)
        else:
            eh = _write_group(vbuf, eh, base)

    smalls = (
        node_features,
        edge_features.reshape(40000, 128),
        edges.reshape(5000, 128),
        node_hidden,
        batch_indices.reshape(1250, 8),
    )
    n = len(smalls)
    vbufs = pl.pallas_call(
        _small_read_body,
        in_specs=[pl.BlockSpec(memory_space=hbm)] * n,
        out_specs=[pl.BlockSpec(memory_space=vmem)] * n,
        out_shape=[jax.ShapeDtypeStruct(a.shape, a.dtype) for a in smalls],
        scratch_shapes=[pltpu.SemaphoreType.DMA((n,))],
    )(*smalls)
    outs = pl.pallas_call(
        _small_read_body,
        in_specs=[pl.BlockSpec(memory_space=vmem)] * n,
        out_specs=[pl.BlockSpec(memory_space=hbm)] * n,
        out_shape=[jax.ShapeDtypeStruct(a.shape, a.dtype) for a in smalls],
        scratch_shapes=[pltpu.SemaphoreType.DMA((n,))],
    )(*vbufs)
    nf, ef, eg, nh, bi = outs
    return (nf, ef.reshape(320000, 16), eg.reshape(2, 320000), nh, eh,
            bi.reshape(10000))


# speedup vs baseline: 1.4118x; 1.4118x over previous
import jax
import jax.numpy as jnp
from jax.experimental import pallas as pl
from jax.experimental.pallas import tpu as pltpu

_C = 4000
_NBUF = 8
_AHEAD = 4


def _eh_copy_body(x_ref, o_ref, buf, in_sems, out_sems):
    n = x_ref.shape[0]
    nchunks = n // _C

    def in_copy(i):
        slot = i % _NBUF
        return pltpu.make_async_copy(
            x_ref.at[pl.ds(i * _C, _C), :], buf.at[slot], in_sems.at[slot])

    def out_copy(i):
        slot = i % _NBUF
        return pltpu.make_async_copy(
            buf.at[slot], o_ref.at[pl.ds(i * _C, _C), :], out_sems.at[slot])

    for j in range(min(_AHEAD, nchunks)):
        in_copy(j).start()
    for i in range(nchunks):
        in_copy(i).wait()
        out_copy(i).start(priority=1)
        j = i + _AHEAD
        if j < nchunks:
            if j >= _NBUF:
                out_copy(j - _NBUF).wait()
            in_copy(j).start()
    for i in range(max(nchunks - _NBUF, 0), nchunks):
        out_copy(i).wait()


def _burst_copy_body(*refs):
    n = len(refs) // 2
    ins, outs, sems = refs[:n], refs[n:2 * n], refs[2 * n]
    for i in range(n):
        pltpu.make_async_copy(ins[i], outs[i], sems.at[i]).start()
    for i in range(n):
        pltpu.make_async_copy(ins[i], outs[i], sems.at[i]).wait()


def kernel(node_features, edge_features, edges, node_hidden, edge_hidden,
           batch_indices, W1, W2, W3, U1, U2):
    hbm = pltpu.MemorySpace.HBM
    vmem = pltpu.MemorySpace.VMEM

    eh = pl.pallas_call(
        _eh_copy_body,
        in_specs=[pl.BlockSpec(memory_space=hbm)],
        out_specs=pl.BlockSpec(memory_space=hbm),
        out_shape=jax.ShapeDtypeStruct(edge_hidden.shape, edge_hidden.dtype),
        scratch_shapes=[
            pltpu.VMEM((_NBUF, _C, 256), jnp.float32),
            pltpu.SemaphoreType.DMA((_NBUF,)),
            pltpu.SemaphoreType.DMA((_NBUF,)),
        ],
    )(edge_hidden)

    smalls = (
        node_features,
        edge_features.reshape(40000, 128),
        edges.reshape(5000, 128),
        node_hidden,
        batch_indices.reshape(1250, 8),
    )
    n = len(smalls)
    vbufs = pl.pallas_call(
        _burst_copy_body,
        in_specs=[pl.BlockSpec(memory_space=hbm)] * n,
        out_specs=[pl.BlockSpec(memory_space=vmem)] * n,
        out_shape=[jax.ShapeDtypeStruct(a.shape, a.dtype) for a in smalls],
        scratch_shapes=[pltpu.SemaphoreType.DMA((n,))],
    )(*smalls)
    outs = pl.pallas_call(
        _burst_copy_body,
        in_specs=[pl.BlockSpec(memory_space=vmem)] * n,
        out_specs=[pl.BlockSpec(memory_space=hbm)] * n,
        out_shape=[jax.ShapeDtypeStruct(a.shape, a.dtype) for a in smalls],
        scratch_shapes=[pltpu.SemaphoreType.DMA((n,))],
    )(*vbufs)
    nf, ef, eg, nh, bi = outs
    return (nf, ef.reshape(320000, 16), eg.reshape(2, 320000), nh, eh,
            bi.reshape(10000))


# speedup vs baseline: 1.4775x; 1.0465x over previous
import jax
import jax.numpy as jnp
from jax.experimental import pallas as pl
from jax.experimental.pallas import tpu as pltpu

_C = 4000
_NBUF = 8
_AHEAD = 4


def _eh_copy_body(x_ref, o_ref, buf, in_sems, out_sems):
    n = x_ref.shape[0]
    nchunks = n // _C

    def in_copy(i):
        slot = i % _NBUF
        return pltpu.make_async_copy(
            x_ref.at[pl.ds(i * _C, _C), :], buf.at[slot], in_sems.at[slot])

    def out_copy(i):
        slot = i % _NBUF
        return pltpu.make_async_copy(
            buf.at[slot], o_ref.at[pl.ds(i * _C, _C), :], out_sems.at[slot])

    for j in range(min(_AHEAD, nchunks)):
        in_copy(j).start()
    for i in range(nchunks):
        in_copy(i).wait()
        out_copy(i).start()
        j = i + _AHEAD
        if j < nchunks:
            if j >= _NBUF:
                out_copy(j - _NBUF).wait()
            in_copy(j).start()
    for i in range(max(nchunks - _NBUF, 0), nchunks):
        out_copy(i).wait()


def _copy_body(x_ref, o_ref):
    o_ref[...] = x_ref[...]


def _copy4_body(a_ref, b_ref, c_ref, d_ref, ao_ref, bo_ref, co_ref, do_ref):
    ao_ref[...] = a_ref[...]
    bo_ref[...] = b_ref[...]
    co_ref[...] = c_ref[...]
    do_ref[...] = d_ref[...]


def _pallas_copy_rows(x, block_rows):
    n, m = x.shape
    return pl.pallas_call(
        _copy_body,
        grid=(n // block_rows,),
        in_specs=[pl.BlockSpec((block_rows, m), lambda i: (i, 0))],
        out_specs=pl.BlockSpec((block_rows, m), lambda i: (i, 0)),
        out_shape=jax.ShapeDtypeStruct(x.shape, x.dtype),
    )(x)


def kernel(node_features, edge_features, edges, node_hidden, edge_hidden,
           batch_indices, W1, W2, W3, U1, U2):
    eh = pl.pallas_call(
        _eh_copy_body,
        in_specs=[pl.BlockSpec(memory_space=pltpu.MemorySpace.HBM)],
        out_specs=pl.BlockSpec(memory_space=pltpu.MemorySpace.HBM),
        out_shape=jax.ShapeDtypeStruct(edge_hidden.shape, edge_hidden.dtype),
        scratch_shapes=[
            pltpu.VMEM((_NBUF, _C, 256), jnp.float32),
            pltpu.SemaphoreType.DMA((_NBUF,)),
            pltpu.SemaphoreType.DMA((_NBUF,)),
        ],
    )(edge_hidden)
    ef = _pallas_copy_rows(edge_features, 16000)
    small = (node_features, edges, node_hidden, batch_indices.reshape(1250, 8))
    nf, eg, nh, bi = pl.pallas_call(
        _copy4_body,
        out_shape=[jax.ShapeDtypeStruct(a.shape, a.dtype) for a in small],
    )(*small)
    return (nf, ef, eg, nh, eh, bi.reshape(10000))
